# baseline (speedup 1.0000x reference)
import jax
import jax.numpy as jnp
from jax import lax
from jax.experimental import pallas as pl
from jax.experimental.pallas import tpu as pltpu

B, N = 1024, 200
BN = B * N
TB = 32
T = TB * N
OFFS = (0, 26, 44, 69, 77)


def _tcol(ci):
    return ((ci >= OFFS[1]).astype(jnp.int32) + (ci >= OFFS[2])
            + (ci >= OFFS[3]) + (ci >= OFFS[4]))


def _body(stT_ref, fT_ref, stWb_ref, W096_ref, A10_ref, Wemb_ref, Tcat_ref,
          b0_ref, out_ref, A3_s, Temb_s, P_s, cmp_s, M_s):
    @pl.when(pl.program_id(0) == 0)
    def _fold():
        A3_s[...] = jnp.dot(stWb_ref[...], W096_ref[...],
                            preferred_element_type=jnp.float32)
        ri = lax.broadcasted_iota(jnp.int32, (80, 80), 0)
        ci = lax.broadcasted_iota(jnp.int32, (80, 80), 1)
        t = _tcol(ri)
        band = (ci >= 16 * t) & (ci < 16 * t + 16)
        Temb_s[...] = jnp.dot(jnp.where(band, Tcat_ref[...], 0.0),
                              Wemb_ref[...], preferred_element_type=jnp.float32)
        ri16 = lax.broadcasted_iota(jnp.int32, (16, 80), 0)
        ci16 = lax.broadcasted_iota(jnp.int32, (16, 80), 1)
        P_s[...] = (ri16 == _tcol(ci16) + 10).astype(jnp.float32)
        ci8 = lax.broadcasted_iota(jnp.int32, (8, 80), 1)
        voff = (26 * (ci8 >= OFFS[1]) + 18 * (ci8 >= OFFS[2])
                + 25 * (ci8 >= OFFS[3]) + 8 * (ci8 >= OFFS[4]))
        cmp_s[...] = (ci8 - voff).astype(jnp.float32)
        M_s[...] = jnp.concatenate(
            [A10_ref[...], jnp.zeros((6, 128), jnp.float32)], axis=0)

    xs = stT_ref[...]
    xf = fT_ref[...]
    dn = (((0,), (0,)), ((), ()))
    u = lax.dot_general(xf, P_s[0:15, :], dn,
                        preferred_element_type=jnp.float32)
    mh = (u == cmp_s[0:1, :]).astype(jnp.float32)
    y = (lax.dot_general(xs, A3_s[0:3, :], dn,
                         preferred_element_type=jnp.float32)
         + lax.dot_general(xf, M_s[0:15, :], dn,
                           preferred_element_type=jnp.float32)
         + jnp.dot(mh, Temb_s[...], preferred_element_type=jnp.float32)
         + A3_s[3:4, :] + b0_ref[...])
    out_ref[...] = jnp.maximum(y, 0.0).reshape(TB, N, 128)


@jax.jit
def kernel(st, feat, st_W, st_b, wind_tab, wth_tab, hour_tab, wk_tab,
           hol_tab, W0, b0):
    stT = st.reshape(BN, 3).T
    fT = feat.reshape(BN, 15).T
    stWb = jnp.concatenate([st_W, st_b[None, :],
                            jnp.zeros((4, 96), jnp.float32)], axis=0)
    W096 = W0[0:96, :]
    A10 = W0[96:106, :]
    Wemb = W0[106:186, :]
    Tcat = jnp.concatenate([wind_tab, wth_tab, hour_tab, wk_tab, hol_tab],
                           axis=0)
    Tcat_rep = jnp.tile(Tcat, (1, 5))
    b0r = b0[None, :]

    out = pl.pallas_call(
        _body,
        grid=(B // TB,),
        in_specs=[
            pl.BlockSpec((3, T), lambda i: (0, i)),
            pl.BlockSpec((15, T), lambda i: (0, i)),
            pl.BlockSpec((8, 96), lambda i: (0, 0)),
            pl.BlockSpec((96, 128), lambda i: (0, 0)),
            pl.BlockSpec((10, 128), lambda i: (0, 0)),
            pl.BlockSpec((80, 128), lambda i: (0, 0)),
            pl.BlockSpec((80, 80), lambda i: (0, 0)),
            pl.BlockSpec((1, 128), lambda i: (0, 0)),
        ],
        out_specs=pl.BlockSpec((TB, N, 128), lambda i: (i, 0, 0)),
        out_shape=jax.ShapeDtypeStruct((B, N, 128), jnp.float32),
        scratch_shapes=[
            pltpu.VMEM((8, 128), jnp.float32),
            pltpu.VMEM((80, 128), jnp.float32),
            pltpu.VMEM((16, 80), jnp.float32),
            pltpu.VMEM((8, 80), jnp.float32),
            pltpu.VMEM((16, 128), jnp.float32),
        ],
        compiler_params=pltpu.CompilerParams(
            dimension_semantics=("arbitrary",)),
    )(stT, fT, stWb, W096, A10, Wemb, Tcat_rep, b0r)
    return out


# speedup vs baseline: 1.0543x; 1.0543x over previous
import jax
import jax.numpy as jnp
from jax import lax
from jax.experimental import pallas as pl
from jax.experimental.pallas import tpu as pltpu

B, N = 1024, 200
BN = B * N
TB = 32
T = TB * N
OFFS = (0, 26, 44, 69, 77)


def _tcol(ci):
    return ((ci >= OFFS[1]).astype(jnp.int32) + (ci >= OFFS[2])
            + (ci >= OFFS[3]) + (ci >= OFFS[4]))


def _body(fT_ref, stWb_ref, W096_ref, A10_ref, Wemb_ref, Tcat_ref,
          b0_ref, out_ref, A3_s, Temb_s, P_s, cmp_s, M_s):
    @pl.when(pl.program_id(0) == 0)
    def _fold():
        A3_s[...] = jnp.dot(stWb_ref[...], W096_ref[...],
                            preferred_element_type=jnp.float32)
        ri = lax.broadcasted_iota(jnp.int32, (80, 80), 0)
        ci = lax.broadcasted_iota(jnp.int32, (80, 80), 1)
        t = _tcol(ri)
        band = (ci >= 16 * t) & (ci < 16 * t + 16)
        Temb_s[...] = jnp.dot(jnp.where(band, Tcat_ref[...], 0.0),
                              Wemb_ref[...], preferred_element_type=jnp.float32)
        ri24 = lax.broadcasted_iota(jnp.int32, (24, 80), 0)
        ci24 = lax.broadcasted_iota(jnp.int32, (24, 80), 1)
        P_s[...] = (ri24 == _tcol(ci24) + 13).astype(jnp.float32)
        ci8 = lax.broadcasted_iota(jnp.int32, (8, 80), 1)
        voff = (26 * (ci8 >= OFFS[1]) + 18 * (ci8 >= OFFS[2])
                + 25 * (ci8 >= OFFS[3]) + 8 * (ci8 >= OFFS[4]))
        cmp_s[...] = (ci8 - voff).astype(jnp.float32)
        M_s[...] = jnp.concatenate(
            [A3_s[0:3, :], A10_ref[...], jnp.zeros((11, 128), jnp.float32)],
            axis=0)

    x = fT_ref[...]
    dn = (((0,), (0,)), ((), ()))
    u = lax.dot_general(x, P_s[0:18, :], dn,
                        preferred_element_type=jnp.float32)
    mh = (u == cmp_s[0:1, :]).astype(jnp.float32)
    y = (lax.dot_general(x, M_s[0:18, :], dn,
                         preferred_element_type=jnp.float32)
         + jnp.dot(mh, Temb_s[...], preferred_element_type=jnp.float32)
         + A3_s[3:4, :] + b0_ref[...])
    out_ref[...] = jnp.maximum(y, 0.0).reshape(TB, N, 128)


@jax.jit
def kernel(st, feat, st_W, st_b, wind_tab, wth_tab, hour_tab, wk_tab,
           hol_tab, W0, b0):
    fT = jnp.concatenate([st.reshape(BN, 3), feat.reshape(BN, 15)],
                         axis=1).T
    stWb = jnp.concatenate([st_W, st_b[None, :],
                            jnp.zeros((4, 96), jnp.float32)], axis=0)
    W096 = W0[0:96, :]
    A10 = W0[96:106, :]
    Wemb = W0[106:186, :]
    Tcat = jnp.concatenate([wind_tab, wth_tab, hour_tab, wk_tab, hol_tab],
                           axis=0)
    Tcat_rep = jnp.tile(Tcat, (1, 5))
    b0r = b0[None, :]

    out = pl.pallas_call(
        _body,
        grid=(B // TB,),
        in_specs=[
            pl.BlockSpec((18, T), lambda i: (0, i)),
            pl.BlockSpec((8, 96), lambda i: (0, 0)),
            pl.BlockSpec((96, 128), lambda i: (0, 0)),
            pl.BlockSpec((10, 128), lambda i: (0, 0)),
            pl.BlockSpec((80, 128), lambda i: (0, 0)),
            pl.BlockSpec((80, 80), lambda i: (0, 0)),
            pl.BlockSpec((1, 128), lambda i: (0, 0)),
        ],
        out_specs=pl.BlockSpec((TB, N, 128), lambda i: (i, 0, 0)),
        out_shape=jax.ShapeDtypeStruct((B, N, 128), jnp.float32),
        scratch_shapes=[
            pltpu.VMEM((8, 128), jnp.float32),
            pltpu.VMEM((80, 128), jnp.float32),
            pltpu.VMEM((24, 80), jnp.float32),
            pltpu.VMEM((8, 80), jnp.float32),
            pltpu.VMEM((24, 128), jnp.float32),
        ],
        compiler_params=pltpu.CompilerParams(
            dimension_semantics=("arbitrary",)),
    )(fT, stWb, W096, A10, Wemb, Tcat_rep, b0r)
    return out


# speedup vs baseline: 1.1044x; 1.0475x over previous
import jax
import jax.numpy as jnp
from jax import lax
from jax.experimental import pallas as pl
from jax.experimental.pallas import tpu as pltpu

B, N = 1024, 200
BN = B * N
TB = 32
T = TB * N
OFFS = (0, 26, 44, 69, 77)


def _tcol(ci):
    return ((ci >= OFFS[1]).astype(jnp.int32) + (ci >= OFFS[2])
            + (ci >= OFFS[3]) + (ci >= OFFS[4]))


def _body(fT_ref, stWb_ref, W096_ref, A10_ref, Wemb_ref, Tcat_ref,
          b0_ref, out_ref, A3_s, Temb_s, P_s, cmp_s, M_s):
    @pl.when(pl.program_id(0) == 0)
    def _fold():
        A3_s[...] = jnp.dot(stWb_ref[...], W096_ref[...],
                            preferred_element_type=jnp.float32)
        ri = lax.broadcasted_iota(jnp.int32, (80, 80), 0)
        ci = lax.broadcasted_iota(jnp.int32, (80, 80), 1)
        t = _tcol(ri)
        band = (ci >= 16 * t) & (ci < 16 * t + 16)
        Temb_s[...] = jnp.dot(jnp.where(band, Tcat_ref[...], 0.0),
                              Wemb_ref[...], preferred_element_type=jnp.float32)
        ri24 = lax.broadcasted_iota(jnp.int32, (24, 80), 0)
        ci24 = lax.broadcasted_iota(jnp.int32, (24, 80), 1)
        P_s[...] = (ri24 == _tcol(ci24) + 13).astype(jnp.float32)
        ci8 = lax.broadcasted_iota(jnp.int32, (8, 80), 1)
        voff = (26 * (ci8 >= OFFS[1]) + 18 * (ci8 >= OFFS[2])
                + 25 * (ci8 >= OFFS[3]) + 8 * (ci8 >= OFFS[4]))
        cmp_s[...] = (ci8 - voff).astype(jnp.float32)
        M_s[...] = jnp.concatenate(
            [A3_s[0:3, :], A10_ref[...], jnp.zeros((11, 128), jnp.float32)],
            axis=0)

    x = fT_ref[...].astype(jnp.float32)
    dn = (((0,), (0,)), ((), ()))
    u = lax.dot_general(x, P_s[0:18, :], dn,
                        preferred_element_type=jnp.float32)
    mh = (u == cmp_s[0:1, :]).astype(jnp.float32)
    y = (lax.dot_general(x, M_s[0:18, :], dn,
                         preferred_element_type=jnp.float32)
         + jnp.dot(mh, Temb_s[...], preferred_element_type=jnp.float32)
         + A3_s[3:4, :] + b0_ref[...])
    out_ref[...] = jnp.maximum(y, 0.0).reshape(TB, N, 128)


@jax.jit
def kernel(st, feat, st_W, st_b, wind_tab, wth_tab, hour_tab, wk_tab,
           hol_tab, W0, b0):
    fT = jnp.concatenate([st.reshape(BN, 3), feat.reshape(BN, 15)],
                         axis=1).T.astype(jnp.bfloat16)
    stWb = jnp.concatenate([st_W, st_b[None, :],
                            jnp.zeros((4, 96), jnp.float32)], axis=0)
    W096 = W0[0:96, :]
    A10 = W0[96:106, :]
    Wemb = W0[106:186, :]
    Tcat = jnp.concatenate([wind_tab, wth_tab, hour_tab, wk_tab, hol_tab],
                           axis=0)
    Tcat_rep = jnp.tile(Tcat, (1, 5))
    b0r = b0[None, :]

    out = pl.pallas_call(
        _body,
        grid=(B // TB,),
        in_specs=[
            pl.BlockSpec((18, T), lambda i: (0, i)),
            pl.BlockSpec((8, 96), lambda i: (0, 0)),
            pl.BlockSpec((96, 128), lambda i: (0, 0)),
            pl.BlockSpec((10, 128), lambda i: (0, 0)),
            pl.BlockSpec((80, 128), lambda i: (0, 0)),
            pl.BlockSpec((80, 80), lambda i: (0, 0)),
            pl.BlockSpec((1, 128), lambda i: (0, 0)),
        ],
        out_specs=pl.BlockSpec((TB, N, 128), lambda i: (i, 0, 0)),
        out_shape=jax.ShapeDtypeStruct((B, N, 128), jnp.float32),
        scratch_shapes=[
            pltpu.VMEM((8, 128), jnp.float32),
            pltpu.VMEM((80, 128), jnp.float32),
            pltpu.VMEM((24, 80), jnp.float32),
            pltpu.VMEM((8, 80), jnp.float32),
            pltpu.VMEM((24, 128), jnp.float32),
        ],
        compiler_params=pltpu.CompilerParams(
            dimension_semantics=("arbitrary",)),
    )(fT, stWb, W096, A10, Wemb, Tcat_rep, b0r)
    return out


# speedup vs baseline: 1.1425x; 1.0345x over previous
import jax
import jax.numpy as jnp
from jax import lax
from jax.experimental import pallas as pl
from jax.experimental.pallas import tpu as pltpu

B, N = 1024, 200
BN = B * N
TB = 64
T = TB * N
OFFS = (0, 26, 44, 69, 77)


def _tcol(ci):
    return ((ci >= OFFS[1]).astype(jnp.int32) + (ci >= OFFS[2])
            + (ci >= OFFS[3]) + (ci >= OFFS[4]))


def _body(fT_ref, stWb_ref, W096_ref, A10_ref, Wemb_ref, Tcat_ref,
          b0_ref, out_ref, A3_s, Temb_s, P_s, cmp_s, M_s):
    @pl.when(pl.program_id(0) == 0)
    def _fold():
        A3_s[...] = jnp.dot(stWb_ref[...], W096_ref[...],
                            preferred_element_type=jnp.float32)
        ri = lax.broadcasted_iota(jnp.int32, (80, 80), 0)
        ci = lax.broadcasted_iota(jnp.int32, (80, 80), 1)
        t = _tcol(ri)
        band = (ci >= 16 * t) & (ci < 16 * t + 16)
        Temb_s[...] = jnp.dot(jnp.where(band, Tcat_ref[...], 0.0),
                              Wemb_ref[...], preferred_element_type=jnp.float32)
        ri24 = lax.broadcasted_iota(jnp.int32, (24, 80), 0)
        ci24 = lax.broadcasted_iota(jnp.int32, (24, 80), 1)
        P_s[...] = (ri24 == _tcol(ci24) + 13).astype(jnp.float32)
        ci8 = lax.broadcasted_iota(jnp.int32, (8, 80), 1)
        voff = (26 * (ci8 >= OFFS[1]) + 18 * (ci8 >= OFFS[2])
                + 25 * (ci8 >= OFFS[3]) + 8 * (ci8 >= OFFS[4]))
        cmp_s[...] = (ci8 - voff).astype(jnp.float32)
        M_s[...] = jnp.concatenate(
            [A3_s[0:3, :], A10_ref[...], jnp.zeros((11, 128), jnp.float32)],
            axis=0)

    x = fT_ref[...].astype(jnp.float32)
    dn = (((0,), (0,)), ((), ()))
    u = lax.dot_general(x, P_s[0:18, :], dn,
                        preferred_element_type=jnp.float32)
    mh = (u == cmp_s[0:1, :]).astype(jnp.float32)
    y = (lax.dot_general(x, M_s[0:18, :], dn,
                         preferred_element_type=jnp.float32)
         + jnp.dot(mh, Temb_s[...], preferred_element_type=jnp.float32)
         + A3_s[3:4, :] + b0_ref[...])
    out_ref[...] = jnp.maximum(y, 0.0).reshape(TB, N, 128)


@jax.jit
def kernel(st, feat, st_W, st_b, wind_tab, wth_tab, hour_tab, wk_tab,
           hol_tab, W0, b0):
    fT = jnp.concatenate([st.reshape(BN, 3), feat.reshape(BN, 15)],
                         axis=1).T.astype(jnp.bfloat16)
    stWb = jnp.concatenate([st_W, st_b[None, :],
                            jnp.zeros((4, 96), jnp.float32)], axis=0)
    W096 = W0[0:96, :]
    A10 = W0[96:106, :]
    Wemb = W0[106:186, :]
    Tcat = jnp.concatenate([wind_tab, wth_tab, hour_tab, wk_tab, hol_tab],
                           axis=0)
    Tcat_rep = jnp.tile(Tcat, (1, 5))
    b0r = b0[None, :]

    out = pl.pallas_call(
        _body,
        grid=(B // TB,),
        in_specs=[
            pl.BlockSpec((18, T), lambda i: (0, i)),
            pl.BlockSpec((8, 96), lambda i: (0, 0)),
            pl.BlockSpec((96, 128), lambda i: (0, 0)),
            pl.BlockSpec((10, 128), lambda i: (0, 0)),
            pl.BlockSpec((80, 128), lambda i: (0, 0)),
            pl.BlockSpec((80, 80), lambda i: (0, 0)),
            pl.BlockSpec((1, 128), lambda i: (0, 0)),
        ],
        out_specs=pl.BlockSpec((TB, N, 128), lambda i: (i, 0, 0)),
        out_shape=jax.ShapeDtypeStruct((B, N, 128), jnp.float32),
        scratch_shapes=[
            pltpu.VMEM((8, 128), jnp.float32),
            pltpu.VMEM((80, 128), jnp.float32),
            pltpu.VMEM((24, 80), jnp.float32),
            pltpu.VMEM((8, 80), jnp.float32),
            pltpu.VMEM((24, 128), jnp.float32),
        ],
        compiler_params=pltpu.CompilerParams(
            dimension_semantics=("arbitrary",)),
    )(fT, stWb, W096, A10, Wemb, Tcat_rep, b0r)
    return out


# speedup vs baseline: 1.1496x; 1.0062x over previous
import jax
import jax.numpy as jnp
from jax import lax
from jax.experimental import pallas as pl
from jax.experimental.pallas import tpu as pltpu

B, N = 1024, 200
BN = B * N
TB = 128
T = TB * N
OFFS = (0, 26, 44, 69, 77)


def _tcol(ci):
    return ((ci >= OFFS[1]).astype(jnp.int32) + (ci >= OFFS[2])
            + (ci >= OFFS[3]) + (ci >= OFFS[4]))


def _body(fT_ref, stWb_ref, W096_ref, A10_ref, Wemb_ref, Tcat_ref,
          b0_ref, out_ref, A3_s, Temb_s, P_s, cmp_s, M_s):
    @pl.when(pl.program_id(0) == 0)
    def _fold():
        A3_s[...] = jnp.dot(stWb_ref[...], W096_ref[...],
                            preferred_element_type=jnp.float32)
        ri = lax.broadcasted_iota(jnp.int32, (80, 80), 0)
        ci = lax.broadcasted_iota(jnp.int32, (80, 80), 1)
        t = _tcol(ri)
        band = (ci >= 16 * t) & (ci < 16 * t + 16)
        Temb_s[...] = jnp.dot(jnp.where(band, Tcat_ref[...], 0.0),
                              Wemb_ref[...], preferred_element_type=jnp.float32)
        ri24 = lax.broadcasted_iota(jnp.int32, (24, 80), 0)
        ci24 = lax.broadcasted_iota(jnp.int32, (24, 80), 1)
        P_s[...] = (ri24 == _tcol(ci24) + 13).astype(jnp.float32)
        ci8 = lax.broadcasted_iota(jnp.int32, (8, 80), 1)
        voff = (26 * (ci8 >= OFFS[1]) + 18 * (ci8 >= OFFS[2])
                + 25 * (ci8 >= OFFS[3]) + 8 * (ci8 >= OFFS[4]))
        cmp_s[...] = (ci8 - voff).astype(jnp.float32)
        M_s[...] = jnp.concatenate(
            [A3_s[0:3, :], A10_ref[...], jnp.zeros((11, 128), jnp.float32)],
            axis=0)

    x = fT_ref[...].astype(jnp.float32)
    dn = (((0,), (0,)), ((), ()))
    u = lax.dot_general(x, P_s[0:18, :], dn,
                        preferred_element_type=jnp.float32)
    mh = (u == cmp_s[0:1, :]).astype(jnp.float32)
    y = (lax.dot_general(x, M_s[0:18, :], dn,
                         preferred_element_type=jnp.float32)
         + jnp.dot(mh, Temb_s[...], preferred_element_type=jnp.float32)
         + A3_s[3:4, :] + b0_ref[...])
    out_ref[...] = jnp.maximum(y, 0.0).reshape(TB, N, 128)


@jax.jit
def kernel(st, feat, st_W, st_b, wind_tab, wth_tab, hour_tab, wk_tab,
           hol_tab, W0, b0):
    fT = jnp.concatenate([st.reshape(BN, 3), feat.reshape(BN, 15)],
                         axis=1).T.astype(jnp.bfloat16)
    stWb = jnp.concatenate([st_W, st_b[None, :],
                            jnp.zeros((4, 96), jnp.float32)], axis=0)
    W096 = W0[0:96, :]
    A10 = W0[96:106, :]
    Wemb = W0[106:186, :]
    Tcat = jnp.concatenate([wind_tab, wth_tab, hour_tab, wk_tab, hol_tab],
                           axis=0)
    Tcat_rep = jnp.tile(Tcat, (1, 5))
    b0r = b0[None, :]

    out = pl.pallas_call(
        _body,
        grid=(B // TB,),
        in_specs=[
            pl.BlockSpec((18, T), lambda i: (0, i)),
            pl.BlockSpec((8, 96), lambda i: (0, 0)),
            pl.BlockSpec((96, 128), lambda i: (0, 0)),
            pl.BlockSpec((10, 128), lambda i: (0, 0)),
            pl.BlockSpec((80, 128), lambda i: (0, 0)),
            pl.BlockSpec((80, 80), lambda i: (0, 0)),
            pl.BlockSpec((1, 128), lambda i: (0, 0)),
        ],
        out_specs=pl.BlockSpec((TB, N, 128), lambda i: (i, 0, 0)),
        out_shape=jax.ShapeDtypeStruct((B, N, 128), jnp.float32),
        scratch_shapes=[
            pltpu.VMEM((8, 128), jnp.float32),
            pltpu.VMEM((80, 128), jnp.float32),
            pltpu.VMEM((24, 80), jnp.float32),
            pltpu.VMEM((8, 80), jnp.float32),
            pltpu.VMEM((24, 128), jnp.float32),
        ],
        compiler_params=pltpu.CompilerParams(
            dimension_semantics=("arbitrary",)),
    )(fT, stWb, W096, A10, Wemb, Tcat_rep, b0r)
    return out
